# baseline (speedup 1.0000x reference)
import jax
import jax.numpy as jnp
from jax.experimental import pallas as pl
from jax.experimental.pallas import tpu as pltpu

_N_CODES = 1024
_D = 256
_BT = 512
_N_TOK = 32768
_GRID = _N_TOK // _BT
_BETA = 0.25


def _vq_tile(zr_ref, zl_ref, e_ref, e2_ref, out_ref, idx_ref, loss_ref,
             acc_ref):
    zr = zr_ref[...]
    e = e_ref[...]
    s = jax.lax.dot_general(
        zr, e, (((1,), (1,)), ((), ())),
        preferred_element_type=jnp.float32)
    z2 = jnp.sum(zr * zr, axis=1, keepdims=True)
    e2 = e2_ref[0:1, :]
    d = (z2 + e2) - 2.0 * s
    m = jnp.min(d, axis=1, keepdims=True)
    iota = jax.lax.broadcasted_iota(jnp.int32, (_BT, _N_CODES), 1)
    cand = jnp.where(d == m, iota, _N_CODES)
    idx = jnp.min(cand, axis=1).astype(jnp.int32)
    idxc = idx.reshape(_BT, 1)
    oh = (iota == idxc).astype(jnp.float32)
    g = jax.lax.dot_general(
        oh, e, (((1,), (0,)), ((), ())),
        preferred_element_type=jnp.float32,
        precision=jax.lax.Precision.HIGHEST)
    out_ref[...] = g
    idx_ref[0, 0, :] = idx

    t = pl.program_id(0)

    @pl.when(t == 0)
    def _init():
        acc_ref[0] = 0.0

    diff = g - zl_ref[...]
    acc_ref[0] += jnp.sum(diff * diff)

    @pl.when(t == _GRID - 1)
    def _fin():
        val = acc_ref[0] * ((1.0 + _BETA) / (_N_TOK * _D))
        loss_ref[...] = jnp.full((1, 1), val, dtype=jnp.float32)


def kernel(z, embedding):
    zrow = jnp.transpose(z, (0, 2, 3, 4, 1)).reshape(_N_TOK, _D)
    zl = z.reshape(_N_TOK, _D)
    e2 = jnp.sum(embedding ** 2, axis=1)
    e2b = jnp.broadcast_to(e2[None, :], (8, _N_CODES))

    out_flat, idx3, loss = pl.pallas_call(
        _vq_tile,
        grid=(_GRID,),
        in_specs=[
            pl.BlockSpec((_BT, _D), lambda t: (t, 0)),
            pl.BlockSpec((_BT, _D), lambda t: (t, 0)),
            pl.BlockSpec((_N_CODES, _D), lambda t: (0, 0)),
            pl.BlockSpec((8, _N_CODES), lambda t: (0, 0)),
        ],
        out_specs=[
            pl.BlockSpec((_BT, _D), lambda t: (t, 0)),
            pl.BlockSpec((1, 1, _BT), lambda t: (t, 0, 0)),
            pl.BlockSpec((1, 1), lambda t: (0, 0)),
        ],
        out_shape=[
            jax.ShapeDtypeStruct((_N_TOK, _D), jnp.float32),
            jax.ShapeDtypeStruct((_GRID, 1, _BT), jnp.int32),
            jax.ShapeDtypeStruct((1, 1), jnp.float32),
        ],
        scratch_shapes=[pltpu.SMEM((1,), jnp.float32)],
    )(zrow, zl, embedding, e2b)

    z_q_out = out_flat.reshape(z.shape)
    encoding_indices = idx3.reshape(_N_TOK)
    vq_loss = loss.reshape(())
    return (z_q_out, vq_loss, encoding_indices)


# speedup vs baseline: 1.2175x; 1.2175x over previous
import jax
import jax.numpy as jnp
from jax.experimental import pallas as pl
from jax.experimental.pallas import tpu as pltpu

_N_CODES = 1024
_D = 256
_BT = 512
_N_TOK = 32768
_GRID = _N_TOK // _BT
_BETA = 0.25


def _vq_tile(zr_ref, zl_ref, e_ref, e2_ref, out_ref, idx_ref, loss_ref,
             acc_ref):
    zr = zr_ref[...]
    e = e_ref[...]
    s = jax.lax.dot_general(
        zr, e, (((1,), (1,)), ((), ())),
        preferred_element_type=jnp.float32)
    z2 = jnp.sum(zr * zr, axis=1, keepdims=True)
    e2 = e2_ref[0:1, :]
    d = (z2 + e2) - 2.0 * s
    m = jnp.min(d, axis=1, keepdims=True)
    iota = jax.lax.broadcasted_iota(jnp.int32, (_BT, _N_CODES), 1)
    cand = jnp.where(d == m, iota, _N_CODES)
    idx = jnp.min(cand, axis=1).astype(jnp.int32)
    idxc = idx.reshape(_BT, 1)
    oh = (iota == idxc).astype(jnp.float32)
    g = jax.lax.dot_general(
        oh, e, (((1,), (0,)), ((), ())),
        preferred_element_type=jnp.float32)
    out_ref[...] = g
    idx_ref[0, 0, :] = idx

    t = pl.program_id(0)

    @pl.when(t == 0)
    def _init():
        acc_ref[0] = 0.0

    diff = g - zl_ref[...]
    acc_ref[0] += jnp.sum(diff * diff)

    @pl.when(t == _GRID - 1)
    def _fin():
        val = acc_ref[0] * ((1.0 + _BETA) / (_N_TOK * _D))
        loss_ref[...] = jnp.full((1, 1), val, dtype=jnp.float32)


def kernel(z, embedding):
    zrow = jnp.transpose(z, (0, 2, 3, 4, 1)).reshape(_N_TOK, _D)
    zl = z.reshape(_N_TOK, _D)
    e2 = jnp.sum(embedding ** 2, axis=1)
    e2b = jnp.broadcast_to(e2[None, :], (8, _N_CODES))

    out_flat, idx3, loss = pl.pallas_call(
        _vq_tile,
        grid=(_GRID,),
        in_specs=[
            pl.BlockSpec((_BT, _D), lambda t: (t, 0)),
            pl.BlockSpec((_BT, _D), lambda t: (t, 0)),
            pl.BlockSpec((_N_CODES, _D), lambda t: (0, 0)),
            pl.BlockSpec((8, _N_CODES), lambda t: (0, 0)),
        ],
        out_specs=[
            pl.BlockSpec((_BT, _D), lambda t: (t, 0)),
            pl.BlockSpec((1, 1, _BT), lambda t: (t, 0, 0)),
            pl.BlockSpec((1, 1), lambda t: (0, 0)),
        ],
        out_shape=[
            jax.ShapeDtypeStruct((_N_TOK, _D), jnp.float32),
            jax.ShapeDtypeStruct((_GRID, 1, _BT), jnp.int32),
            jax.ShapeDtypeStruct((1, 1), jnp.float32),
        ],
        scratch_shapes=[pltpu.SMEM((1,), jnp.float32)],
    )(zrow, zl, embedding, e2b)

    z_q_out = out_flat.reshape(z.shape)
    encoding_indices = idx3.reshape(_N_TOK)
    vq_loss = loss.reshape(())
    return (z_q_out, vq_loss, encoding_indices)
